# Initial kernel scaffold; baseline (speedup 1.0000x reference)
import jax
import jax.numpy as jnp
from jax.experimental import pallas as pl


def kernel(x, edge_index, W1, b1, W2, b2, fc1_W, fc1_b, fc2_W, fc2_b):
    raise NotImplementedError("write your pallas kernel here")


# speedup vs baseline: 10.0119x; 10.0119x over previous
import jax
import jax.numpy as jnp
from jax import lax
from jax.experimental import pallas as pl
from jax.experimental.pallas import tpu as pltpu
from jax.experimental.pallas import tpu_sc as plsc

N = 10000
D = 128
E = 320000
NP = 10240
DUMP = N
NC = 2
NS = 16
GPW = 80
G = NC * NS * GPW
EPAD = G * 128
RPS = NP // NS
CHG = 8
F32 = jnp.float32


def _mesh():
    return plsc.VectorSubcoreMesh(core_axis_name="c", subcore_axis_name="s")


def _deg_body(dst_hbm, cnt_hbm, idxv, onesv, zv, cnt_sh):
    c = lax.axis_index("c")
    s = lax.axis_index("s")

    def initrow(i, _):
        onesv[i] = jnp.full((16,), 1.0, F32)
        zv[i] = jnp.zeros((16,), F32)
        return 0

    lax.fori_loop(0, 128, initrow, 0)
    r0 = s * RPS
    for k in range(5):
        pltpu.sync_copy(zv, cnt_sh.at[pl.ds(r0 + k * 128, 128)])
    plsc.subcore_barrier()

    g0 = (c * NS + s) * GPW
    pltpu.sync_copy(dst_hbm.at[pl.ds(g0, GPW)], idxv)

    def group(j, _):
        pltpu.sync_copy(onesv, cnt_sh.at[idxv.at[j]], add=True)
        return 0

    lax.fori_loop(0, GPW, group, 0)
    plsc.subcore_barrier()
    for k in range(5):
        pltpu.sync_copy(cnt_sh.at[pl.ds(r0 + k * 128, 128)],
                        cnt_hbm.at[c, pl.ds(r0 + k * 128, 128)])


_deg_kernel = pl.kernel(
    _deg_body,
    out_type=jax.ShapeDtypeStruct((NC, NP, 16), F32),
    mesh=_mesh(),
    scratch_types=[
        pltpu.VMEM((GPW, 128), jnp.int32),
        pltpu.VMEM((128, 16), F32),
        pltpu.VMEM((128, 16), F32),
        pltpu.VMEM_SHARED((NP, 16), F32),
    ],
)


def _edge_body(tab_hbm, src_hbm, dst_hbm, acc_hbm,
               srcv, dstv, rows0, rows1, acc_sh, sem0, sem1):
    c = lax.axis_index("c")
    s = lax.axis_index("s")

    def zrow(i, _):
        for k in range(8):
            rows0[i, pl.ds(k * 16, 16)] = jnp.zeros((16,), F32)
        return 0

    lax.fori_loop(0, 128, zrow, 0)
    r0 = s * RPS
    for k in range(5):
        pltpu.sync_copy(rows0, acc_sh.at[pl.ds(r0 + k * 128, 128)])
    plsc.subcore_barrier()

    g0 = (c * NS + s) * GPW

    def chunk(t, _):
        pltpu.sync_copy(src_hbm.at[pl.ds(g0 + t * CHG, CHG)], srcv)
        pltpu.sync_copy(dst_hbm.at[pl.ds(g0 + t * CHG, CHG)], dstv)
        pltpu.async_copy(tab_hbm.at[srcv.at[0]], rows0, sem0)
        for p in range(CHG // 2):
            j0 = 2 * p
            pltpu.async_copy(tab_hbm.at[srcv.at[j0 + 1]], rows1, sem1)
            pltpu.make_async_copy(tab_hbm.at[srcv.at[j0]], rows0, sem0).wait()
            pltpu.sync_copy(rows0, acc_sh.at[dstv.at[j0]], add=True)
            if j0 + 2 < CHG:
                pltpu.async_copy(tab_hbm.at[srcv.at[j0 + 2]], rows0, sem0)
            pltpu.make_async_copy(tab_hbm.at[srcv.at[j0 + 1]], rows1,
                                  sem1).wait()
            pltpu.sync_copy(rows1, acc_sh.at[dstv.at[j0 + 1]], add=True)
        return 0

    lax.fori_loop(0, GPW // CHG, chunk, 0)

    plsc.subcore_barrier()
    for k in range(5):
        pltpu.sync_copy(acc_sh.at[pl.ds(r0 + k * 128, 128)],
                        acc_hbm.at[c, pl.ds(r0 + k * 128, 128)])


_edge_kernel = pl.kernel(
    _edge_body,
    out_type=jax.ShapeDtypeStruct((NC, NP, D), F32),
    mesh=_mesh(),
    scratch_types=[
        pltpu.VMEM((CHG, 128), jnp.int32),
        pltpu.VMEM((CHG, 128), jnp.int32),
        pltpu.VMEM((128, D), F32),
        pltpu.VMEM((128, D), F32),
        pltpu.VMEM_SHARED((NP, D), F32),
        pltpu.SemaphoreType.DMA,
        pltpu.SemaphoreType.DMA,
    ],
)


_HI = jax.lax.Precision.HIGHEST


def _m2_body(cnt_ref, x_ref, w1_ref, hs_ref, dinv_ref):
    csum = cnt_ref[0] + cnt_ref[1]
    deg = csum[:, 0:1] + 1.0
    dinv = lax.rsqrt(deg)
    h = jnp.dot(x_ref[...], w1_ref[...], preferred_element_type=F32,
                precision=_HI)
    hs_ref[...] = h * dinv
    dinv_ref[...] = dinv


def _m3_body(acc_ref, hs_ref, dinv_ref, b1_ref, w2_ref, hs2_ref):
    dinv = dinv_ref[...]
    a1 = dinv * (acc_ref[0] + acc_ref[1] + hs_ref[...]) + b1_ref[...]
    a1 = jnp.maximum(a1, 0.0)
    h2 = jnp.dot(a1, w2_ref[...], preferred_element_type=F32, precision=_HI)
    hs2_ref[...] = h2 * dinv


def _m4_body(acc_ref, hs_ref, dinv_ref, b2_ref, fc1w_ref, fc1b_ref,
             fc2w_ref, fc2b_ref, emb_ref, z_ref):
    dinv = dinv_ref[...]
    emb = dinv * (acc_ref[0] + acc_ref[1] + hs_ref[...]) + b2_ref[...]
    emb = jnp.maximum(emb, 0.0)
    t = jnp.dot(emb, fc1w_ref[...], preferred_element_type=F32,
                precision=_HI) + fc1b_ref[...]
    t = jnp.where(t > 0.0, t, jnp.exp(jnp.minimum(t, 0.0)) - 1.0)
    z = jnp.dot(t, fc2w_ref[...], preferred_element_type=F32,
                precision=_HI) + fc2b_ref[...]
    emb_ref[...] = emb
    z_ref[...] = z


_m2 = pl.pallas_call(
    _m2_body,
    out_shape=[jax.ShapeDtypeStruct((NP, D), F32),
               jax.ShapeDtypeStruct((NP, 1), F32)],
)
_m3 = pl.pallas_call(
    _m3_body,
    out_shape=jax.ShapeDtypeStruct((NP, D), F32),
)
_m4 = pl.pallas_call(
    _m4_body,
    out_shape=[jax.ShapeDtypeStruct((NP, D), F32),
               jax.ShapeDtypeStruct((NP, D), F32)],
)


@jax.jit
def kernel(x, edge_index, W1, b1, W2, b2, fc1_W, fc1_b, fc2_W, fc2_b):
    ei = edge_index.astype(jnp.int32)
    pad = jnp.full((2, EPAD - E), DUMP, jnp.int32)
    e = jnp.concatenate([ei, pad], axis=1)
    src2d = e[0].reshape(G, 128)
    dst2d = e[1].reshape(G, 128)
    x_pad = jnp.pad(x, ((0, NP - N), (0, 0)))

    cnt = _deg_kernel(dst2d)
    hs1, dinv = _m2(cnt, x_pad, W1)
    acc1 = _edge_kernel(hs1, src2d, dst2d)
    hs2 = _m3(acc1, hs1, dinv, b1.reshape(1, D), W2)
    acc2 = _edge_kernel(hs2, src2d, dst2d)
    emb, z = _m4(acc2, hs2, dinv, b2.reshape(1, D), fc1_W,
                 fc1_b.reshape(1, D), fc2_W, fc2_b.reshape(1, D))
    return emb[:N], z[:N]


# speedup vs baseline: 10.0150x; 1.0003x over previous
import jax
import jax.numpy as jnp
from jax import lax
from jax.experimental import pallas as pl
from jax.experimental.pallas import tpu as pltpu
from jax.experimental.pallas import tpu_sc as plsc

N = 10000
D = 128
E = 320000
NP = 10240
DUMP = N
NC = 2
NS = 16
GPW = 80
G = NC * NS * GPW
GPW0 = 80
GPW1 = 2 * GPW - GPW0
G0 = NS * GPW0
EPAD = G * 128
RPS = NP // NS
CHG = 8
F32 = jnp.float32


def _mesh():
    return plsc.VectorSubcoreMesh(core_axis_name="c", subcore_axis_name="s")


def _deg_body(dst_hbm, cnt_hbm, idxv, onesv, zv, cnt_sh):
    c = lax.axis_index("c")
    s = lax.axis_index("s")

    def initrow(i, _):
        onesv[i] = jnp.full((16,), 1.0, F32)
        zv[i] = jnp.zeros((16,), F32)
        return 0

    lax.fori_loop(0, 128, initrow, 0)
    r0 = s * RPS
    for k in range(5):
        pltpu.sync_copy(zv, cnt_sh.at[pl.ds(r0 + k * 128, 128)])
    plsc.subcore_barrier()

    g0 = (c * NS + s) * GPW
    pltpu.sync_copy(dst_hbm.at[pl.ds(g0, GPW)], idxv)

    def group(j, _):
        pltpu.sync_copy(onesv, cnt_sh.at[idxv.at[j]], add=True)
        return 0

    lax.fori_loop(0, GPW, group, 0)
    plsc.subcore_barrier()
    for k in range(5):
        pltpu.sync_copy(cnt_sh.at[pl.ds(r0 + k * 128, 128)],
                        cnt_hbm.at[c, pl.ds(r0 + k * 128, 128)])


_deg_kernel = pl.kernel(
    _deg_body,
    out_type=jax.ShapeDtypeStruct((NC, NP, 16), F32),
    mesh=_mesh(),
    scratch_types=[
        pltpu.VMEM((GPW, 128), jnp.int32),
        pltpu.VMEM((128, 16), F32),
        pltpu.VMEM((128, 16), F32),
        pltpu.VMEM_SHARED((NP, 16), F32),
    ],
)


def _edge_body(tab_hbm, src_hbm, dst_hbm, acc_hbm,
               srcv, dstv, rows0, rows1, acc_sh, sem0, sem1):
    c = lax.axis_index("c")
    s = lax.axis_index("s")

    def zrow(i, _):
        for k in range(8):
            rows0[i, pl.ds(k * 16, 16)] = jnp.zeros((16,), F32)
        return 0

    lax.fori_loop(0, 128, zrow, 0)
    r0 = s * RPS
    for k in range(5):
        pltpu.sync_copy(rows0, acc_sh.at[pl.ds(r0 + k * 128, 128)])
    plsc.subcore_barrier()

    def run(g0, nchunks):
        def chunk(t, _):
            pltpu.sync_copy(src_hbm.at[pl.ds(g0 + t * CHG, CHG)], srcv)
            pltpu.sync_copy(dst_hbm.at[pl.ds(g0 + t * CHG, CHG)], dstv)
            pltpu.async_copy(tab_hbm.at[srcv.at[0]], rows0, sem0)
            for p in range(CHG // 2):
                j0 = 2 * p
                pltpu.async_copy(tab_hbm.at[srcv.at[j0 + 1]], rows1, sem1)
                pltpu.make_async_copy(tab_hbm.at[srcv.at[j0]], rows0,
                                      sem0).wait()
                pltpu.sync_copy(rows0, acc_sh.at[dstv.at[j0]], add=True)
                if j0 + 2 < CHG:
                    pltpu.async_copy(tab_hbm.at[srcv.at[j0 + 2]], rows0, sem0)
                pltpu.make_async_copy(tab_hbm.at[srcv.at[j0 + 1]], rows1,
                                      sem1).wait()
                pltpu.sync_copy(rows1, acc_sh.at[dstv.at[j0 + 1]], add=True)
            return 0

        lax.fori_loop(0, nchunks, chunk, 0)

    @pl.when(c == 0)
    def _core0():
        run(s * GPW0, GPW0 // CHG)

    @pl.when(c != 0)
    def _core1():
        run(G0 + s * GPW1, GPW1 // CHG)

    plsc.subcore_barrier()
    for k in range(5):
        pltpu.sync_copy(acc_sh.at[pl.ds(r0 + k * 128, 128)],
                        acc_hbm.at[c, pl.ds(r0 + k * 128, 128)])


_edge_kernel = pl.kernel(
    _edge_body,
    out_type=jax.ShapeDtypeStruct((NC, NP, D), F32),
    mesh=_mesh(),
    scratch_types=[
        pltpu.VMEM((CHG, 128), jnp.int32),
        pltpu.VMEM((CHG, 128), jnp.int32),
        pltpu.VMEM((128, D), F32),
        pltpu.VMEM((128, D), F32),
        pltpu.VMEM_SHARED((NP, D), F32),
        pltpu.SemaphoreType.DMA,
        pltpu.SemaphoreType.DMA,
    ],
)


_HI = jax.lax.Precision.HIGHEST


def _m2_body(cnt_ref, x_ref, w1_ref, hs_ref, dinv_ref):
    csum = cnt_ref[0] + cnt_ref[1]
    deg = csum[:, 0:1] + 1.0
    dinv = lax.rsqrt(deg)
    h = jnp.dot(x_ref[...], w1_ref[...], preferred_element_type=F32,
                precision=_HI)
    hs_ref[...] = h * dinv
    dinv_ref[...] = dinv


def _m3_body(acc_ref, hs_ref, dinv_ref, b1_ref, w2_ref, hs2_ref):
    dinv = dinv_ref[...]
    a1 = dinv * (acc_ref[0] + acc_ref[1] + hs_ref[...]) + b1_ref[...]
    a1 = jnp.maximum(a1, 0.0)
    h2 = jnp.dot(a1, w2_ref[...], preferred_element_type=F32, precision=_HI)
    hs2_ref[...] = h2 * dinv


def _m4_body(acc_ref, hs_ref, dinv_ref, b2_ref, fc1w_ref, fc1b_ref,
             fc2w_ref, fc2b_ref, emb_ref, z_ref):
    dinv = dinv_ref[...]
    emb = dinv * (acc_ref[0] + acc_ref[1] + hs_ref[...]) + b2_ref[...]
    emb = jnp.maximum(emb, 0.0)
    t = jnp.dot(emb, fc1w_ref[...], preferred_element_type=F32,
                precision=_HI) + fc1b_ref[...]
    t = jnp.where(t > 0.0, t, jnp.exp(jnp.minimum(t, 0.0)) - 1.0)
    z = jnp.dot(t, fc2w_ref[...], preferred_element_type=F32,
                precision=_HI) + fc2b_ref[...]
    emb_ref[...] = emb
    z_ref[...] = z


_m2 = pl.pallas_call(
    _m2_body,
    out_shape=[jax.ShapeDtypeStruct((NP, D), F32),
               jax.ShapeDtypeStruct((NP, 1), F32)],
)
_m3 = pl.pallas_call(
    _m3_body,
    out_shape=jax.ShapeDtypeStruct((NP, D), F32),
)
_m4 = pl.pallas_call(
    _m4_body,
    out_shape=[jax.ShapeDtypeStruct((NP, D), F32),
               jax.ShapeDtypeStruct((NP, D), F32)],
)


@jax.jit
def kernel(x, edge_index, W1, b1, W2, b2, fc1_W, fc1_b, fc2_W, fc2_b):
    ei = edge_index.astype(jnp.int32)
    pad = jnp.full((2, EPAD - E), DUMP, jnp.int32)
    e = jnp.concatenate([ei, pad], axis=1)
    src2d = e[0].reshape(G, 128)
    dst2d = e[1].reshape(G, 128)
    x_pad = jnp.pad(x, ((0, NP - N), (0, 0)))

    cnt = _deg_kernel(dst2d)
    hs1, dinv = _m2(cnt, x_pad, W1)
    acc1 = _edge_kernel(hs1, src2d, dst2d)
    hs2 = _m3(acc1, hs1, dinv, b1.reshape(1, D), W2)
    acc2 = _edge_kernel(hs2, src2d, dst2d)
    emb, z = _m4(acc2, hs2, dinv, b2.reshape(1, D), fc1_W,
                 fc1_b.reshape(1, D), fc2_W, fc2_b.reshape(1, D))
    return emb[:N], z[:N]
